# Initial kernel scaffold; baseline (speedup 1.0000x reference)
import jax
import jax.numpy as jnp
from jax.experimental import pallas as pl


def kernel(x, edge_index, W_src, b_src, W_dst, b_dst):
    raise NotImplementedError("write your pallas kernel here")


# speedup vs baseline: 6.2051x; 6.2051x over previous
import functools

import jax
import jax.numpy as jnp
from jax import lax
from jax.experimental import pallas as pl
from jax.experimental.pallas import tpu as pltpu
from jax.experimental.pallas import tpu_sc as plsc

_OUT = 16

_NUM_CORES = 2
_NUM_SUBCORES = 16
_NW = _NUM_CORES * _NUM_SUBCORES


def _proj_body(x_ref, ws_ref, bs_ref, wd_ref, bd_ref, src_ref, dst_ref):
    x = x_ref[...]
    src_ref[...] = (
        jnp.dot(x, ws_ref[...], preferred_element_type=jnp.float32) + bs_ref[...]
    )
    dst_ref[...] = (
        jnp.dot(x, wd_ref[...], preferred_element_type=jnp.float32) + bd_ref[...]
    )


@jax.jit
def _proj(x, W_src, b_src, W_dst, b_dst):
    n = x.shape[0]
    out = jax.ShapeDtypeStruct((n, _OUT), jnp.float32)
    return pl.pallas_call(
        _proj_body,
        out_shape=[out, out],
    )(x, W_src, b_src.reshape(1, _OUT), W_dst, b_dst.reshape(1, _OUT))


def _make_gather(n_edges: int, chunk: int):
    assert n_edges % (_NW * chunk) == 0
    epw = n_edges // _NW
    n_chunks = epw // chunk
    mesh = plsc.VectorSubcoreMesh(core_axis_name="c", subcore_axis_name="s")

    @functools.partial(
        pl.kernel,
        mesh=mesh,
        compiler_params=pltpu.CompilerParams(use_tc_tiling_on_sc=False),
        out_type=jax.ShapeDtypeStruct((n_edges, _OUT), jnp.float32),
        scratch_types=[
            pltpu.VMEM((chunk,), jnp.int32),
            pltpu.VMEM((chunk,), jnp.int32),
            pltpu.VMEM((chunk, _OUT), jnp.float32),
            pltpu.VMEM((chunk, _OUT), jnp.float32),
            pltpu.SemaphoreType.DMA,
            pltpu.SemaphoreType.DMA,
        ],
    )
    def _gather(src_hbm, dst_hbm, u_hbm, v_hbm, out_hbm, u_v, v_v, a_v, b_v, s1, s2):
        wid = lax.axis_index("s") * _NUM_CORES + lax.axis_index("c")
        for g in range(n_chunks):
            base = wid * epw + g * chunk
            pltpu.sync_copy(u_hbm.at[pl.ds(base, chunk)], u_v)
            pltpu.sync_copy(v_hbm.at[pl.ds(base, chunk)], v_v)
            cp1 = pltpu.async_copy(src_hbm.at[u_v], a_v, s1)
            cp2 = pltpu.async_copy(dst_hbm.at[v_v], b_v, s2)
            cp1.wait()
            cp2.wait()

            def _add(i, carry):
                a_v[i] = a_v[i] + b_v[i]
                return carry

            lax.fori_loop(0, chunk, _add, 0, unroll=4)
            pltpu.sync_copy(a_v, out_hbm.at[pl.ds(base, chunk)])

    return _gather


def kernel(x, edge_index, W_src, b_src, W_dst, b_dst):
    src_tab, dst_tab = _proj(x, W_src, b_src, W_dst, b_dst)
    u = edge_index[0].astype(jnp.int32)
    v = edge_index[1].astype(jnp.int32)
    n_edges = u.shape[0]
    gather = _make_gather(n_edges, chunk=2000)
    return gather(src_tab, dst_tab, u, v)


# speedup vs baseline: 7.9669x; 1.2839x over previous
import functools

import jax
import jax.numpy as jnp
from jax import lax
from jax.experimental import pallas as pl
from jax.experimental.pallas import tpu as pltpu
from jax.experimental.pallas import tpu_sc as plsc

_OUT = 16

_NUM_CORES = 2
_NUM_SUBCORES = 16
_NW = _NUM_CORES * _NUM_SUBCORES
_NSLOT = 3


def _proj_body(x_ref, ws_ref, bs_ref, wd_ref, bd_ref, src_ref, dst_ref):
    x = x_ref[...]
    src_ref[...] = (
        jnp.dot(x, ws_ref[...], preferred_element_type=jnp.float32) + bs_ref[...]
    )
    dst_ref[...] = (
        jnp.dot(x, wd_ref[...], preferred_element_type=jnp.float32) + bd_ref[...]
    )


@jax.jit
def _proj(x, W_src, b_src, W_dst, b_dst):
    n = x.shape[0]
    out = jax.ShapeDtypeStruct((n, _OUT), jnp.float32)
    return pl.pallas_call(
        _proj_body,
        out_shape=[out, out],
    )(x, W_src, b_src.reshape(1, _OUT), W_dst, b_dst.reshape(1, _OUT))


def _make_gather(n_edges: int, chunk: int):
    assert n_edges % (_NW * chunk) == 0
    epw = n_edges // _NW
    n_chunks = epw // chunk
    mesh = plsc.VectorSubcoreMesh(core_axis_name="c", subcore_axis_name="s")

    @functools.partial(
        pl.kernel,
        mesh=mesh,
        compiler_params=pltpu.CompilerParams(use_tc_tiling_on_sc=False),
        out_type=jax.ShapeDtypeStruct((n_edges, _OUT), jnp.float32),
        scratch_types=[
            pltpu.VMEM((epw,), jnp.int32),
            pltpu.VMEM((epw,), jnp.int32),
        ]
        + [pltpu.VMEM((chunk, _OUT), jnp.float32) for _ in range(_NSLOT)]
        + [pltpu.SemaphoreType.DMA for _ in range(1 + 3 * _NSLOT)],
    )
    def _gather(src_hbm, dst_hbm, ei_hbm, out_hbm, u_all, v_all, *rest):
        bufs = list(rest[:_NSLOT])
        si = rest[_NSLOT]
        sg1 = list(rest[_NSLOT + 1 : _NSLOT + 1 + _NSLOT])
        sg2 = list(rest[_NSLOT + 1 + _NSLOT : _NSLOT + 1 + 2 * _NSLOT])
        sst = list(rest[_NSLOT + 1 + 2 * _NSLOT :])

        wid = lax.axis_index("s") * _NUM_CORES + lax.axis_index("c")
        base0 = wid * epw
        cu = pltpu.async_copy(ei_hbm.at[0, pl.ds(base0, epw)], u_all, si)
        cv = pltpu.async_copy(ei_hbm.at[1, pl.ds(base0, epw)], v_all, si)
        cu.wait()
        cv.wait()

        g1 = [None] * n_chunks
        g2 = [None] * n_chunks
        st = [None] * n_chunks
        for k in range(n_chunks + 2):
            if 1 <= k <= n_chunks:
                kk = k - 1
                s = kk % _NSLOT
                g1[kk].wait()
                g2[kk] = pltpu.async_copy(
                    dst_hbm.at[v_all.at[pl.ds(kk * chunk, chunk)]],
                    bufs[s],
                    sg2[s],
                    add=True,
                )
            if 2 <= k <= n_chunks + 1:
                kk = k - 2
                s = kk % _NSLOT
                g2[kk].wait()
                st[kk] = pltpu.async_copy(
                    bufs[s],
                    out_hbm.at[pl.ds(base0 + kk * chunk, chunk)],
                    sst[s],
                )
            if k < n_chunks:
                s = k % _NSLOT
                if k >= _NSLOT:
                    st[k - _NSLOT].wait()
                g1[k] = pltpu.async_copy(
                    src_hbm.at[u_all.at[pl.ds(k * chunk, chunk)]],
                    bufs[s],
                    sg1[s],
                )
        for kk in range(max(0, n_chunks - _NSLOT), n_chunks):
            st[kk].wait()

    return _gather


def kernel(x, edge_index, W_src, b_src, W_dst, b_dst):
    src_tab, dst_tab = _proj(x, W_src, b_src, W_dst, b_dst)
    ei = edge_index.astype(jnp.int32)
    n_edges = ei.shape[1]
    gather = _make_gather(n_edges, chunk=2000)
    return gather(src_tab, dst_tab, ei)


# speedup vs baseline: 11.5756x; 1.4530x over previous
import functools

import jax
import jax.numpy as jnp
from jax import lax
from jax.experimental import pallas as pl
from jax.experimental.pallas import tpu as pltpu
from jax.experimental.pallas import tpu_sc as plsc

_OUT = 16
_LANE = 16

_NUM_CORES = 2
_NUM_SUBCORES = 16
_NW = _NUM_CORES * _NUM_SUBCORES
_CHUNK_TILES = 10
_TILE = 128


def _proj_body(x_ref, ws_ref, bs_ref, wd_ref, bd_ref, src_ref, dst_ref):
    x = x_ref[...]
    src_ref[...] = (
        jnp.dot(x, ws_ref[...], preferred_element_type=jnp.float32) + bs_ref[...]
    )
    dst_ref[...] = (
        jnp.dot(x, wd_ref[...], preferred_element_type=jnp.float32) + bd_ref[...]
    )


@jax.jit
def _proj(x, W_src, b_src, W_dst, b_dst):
    n = x.shape[0]
    out = jax.ShapeDtypeStruct((n, _OUT), jnp.float32)
    return pl.pallas_call(
        _proj_body,
        out_shape=[out, out],
    )(x, W_src, b_src.reshape(1, _OUT), W_dst, b_dst.reshape(1, _OUT))


def _make_gather(n_edges: int):
    assert n_edges % _TILE == 0
    n_tiles = n_edges // _TILE
    tw = -(-n_tiles // _NW)
    tw = -(-tw // _CHUNK_TILES) * _CHUNK_TILES
    n_chunks = tw // _CHUNK_TILES
    chunk = _CHUNK_TILES * _TILE
    epw = tw * _TILE
    groups = chunk // _LANE

    mesh = plsc.VectorSubcoreMesh(core_axis_name="c", subcore_axis_name="s")

    @functools.partial(
        pl.kernel,
        mesh=mesh,
        compiler_params=pltpu.CompilerParams(
            use_tc_tiling_on_sc=False, needs_layout_passes=False
        ),
        out_type=jax.ShapeDtypeStruct((2, n_tiles, _OUT // 2, _TILE), jnp.float32),
        scratch_types=[
            pltpu.VMEM((epw,), jnp.int32),
            pltpu.VMEM((epw,), jnp.int32),
        ]
        + [pltpu.VMEM((chunk, _OUT), jnp.float32) for _ in range(2)]
        + [pltpu.VMEM((2, _CHUNK_TILES, _OUT // 2, _TILE), jnp.float32) for _ in range(2)]
        + [pltpu.SemaphoreType.DMA for _ in range(7)],
    )
    def _gather(src_hbm, dst_hbm, ei_hbm, out_hbm, u_all, v_all, a0, a1, t0, t1, *sems):
        a_v = [a0, a1]
        t_v = [t0, t1]
        si = sems[0]
        sg1 = list(sems[1:3])
        sg2 = list(sems[3:5])
        sst = list(sems[5:7])

        wid = lax.axis_index("s") * _NUM_CORES + lax.axis_index("c")
        tile_lo = jnp.minimum(wid * n_tiles // _NW, n_tiles - tw)
        base0 = tile_lo * _TILE

        cu = pltpu.async_copy(ei_hbm.at[0, pl.ds(base0, epw)], u_all, si)
        cv = pltpu.async_copy(ei_hbm.at[1, pl.ds(base0, epw)], v_all, si)
        cu.wait()
        cv.wait()

        iota = lax.iota(jnp.int32, _LANE)
        fcols = [jnp.full((_LANE,), f, jnp.int32) for f in range(_OUT)]

        def _transpose(a_ref, t_ref):
            def body(g, carry):
                row_idx = g * _LANE + iota
                tile = g // (_TILE // _LANE)
                e_off = (g % (_TILE // _LANE)) * _LANE
                for f in range(_OUT):
                    vec = plsc.load_gather(a_ref, [row_idx, fcols[f]])
                    t_ref[f // 8, tile, f % 8, pl.ds(e_off, _LANE)] = vec
                return carry

            lax.fori_loop(0, groups, body, 0)

        g1 = [None] * n_chunks
        g2 = [None] * n_chunks
        st = [None] * n_chunks

        g1[0] = pltpu.async_copy(
            src_hbm.at[u_all.at[pl.ds(0, chunk)]], a_v[0], sg1[0]
        )
        for k in range(n_chunks):
            s = k % 2
            g1[k].wait()
            g2[k] = pltpu.async_copy(
                dst_hbm.at[v_all.at[pl.ds(k * chunk, chunk)]],
                a_v[s],
                sg2[s],
                add=True,
            )
            if k + 1 < n_chunks:
                g1[k + 1] = pltpu.async_copy(
                    src_hbm.at[u_all.at[pl.ds((k + 1) * chunk, chunk)]],
                    a_v[(k + 1) % 2],
                    sg1[(k + 1) % 2],
                )
            g2[k].wait()
            if k >= 2:
                st[k - 2].wait()
            _transpose(a_v[s], t_v[s])
            st[k] = pltpu.async_copy(
                t_v[s],
                out_hbm.at[:, pl.ds(tile_lo + k * _CHUNK_TILES, _CHUNK_TILES)],
                sst[s],
            )
        st[n_chunks - 2].wait()
        st[n_chunks - 1].wait()

    return _gather


def kernel(x, edge_index, W_src, b_src, W_dst, b_dst):
    src_tab, dst_tab = _proj(x, W_src, b_src, W_dst, b_dst)
    ei = edge_index.astype(jnp.int32)
    n_edges = ei.shape[1]
    gather = _make_gather(n_edges)
    v = gather(src_tab, dst_tab, ei)
    return v.transpose(1, 3, 0, 2).reshape(n_edges, _OUT)


# speedup vs baseline: 22.8899x; 1.9774x over previous
import functools

import jax
import jax.numpy as jnp
from jax import lax
from jax.experimental import pallas as pl
from jax.experimental.pallas import tpu as pltpu
from jax.experimental.pallas import tpu_sc as plsc

_OUT = 16
_LANE = 16

_NUM_CORES = 2
_NUM_SUBCORES = 16
_NW = _NUM_CORES * _NUM_SUBCORES
_CHUNK_TILES = 10
_TILE = 128


def _proj_body(x_ref, ws_ref, bs_ref, wd_ref, bd_ref, src_ref, dst_ref):
    x = x_ref[...]
    src_ref[...] = (
        jnp.dot(x, ws_ref[...], preferred_element_type=jnp.float32) + bs_ref[...]
    )
    dst_ref[...] = (
        jnp.dot(x, wd_ref[...], preferred_element_type=jnp.float32) + bd_ref[...]
    )


@jax.jit
def _proj(x, W_src, b_src, W_dst, b_dst):
    n = x.shape[0]
    out = jax.ShapeDtypeStruct((n, _OUT), jnp.float32)
    return pl.pallas_call(
        _proj_body,
        out_shape=[out, out],
    )(x, W_src, b_src.reshape(1, _OUT), W_dst, b_dst.reshape(1, _OUT))


def _make_gather(n_edges: int):
    assert n_edges % _TILE == 0
    n_tiles = n_edges // _TILE
    tw = -(-n_tiles // _NW)
    tw = -(-tw // _CHUNK_TILES) * _CHUNK_TILES
    n_chunks = tw // _CHUNK_TILES
    chunk = _CHUNK_TILES * _TILE
    epw = tw * _TILE
    groups = chunk // _LANE

    mesh = plsc.VectorSubcoreMesh(core_axis_name="c", subcore_axis_name="s")

    @functools.partial(
        pl.kernel,
        mesh=mesh,
        compiler_params=pltpu.CompilerParams(
            use_tc_tiling_on_sc=False, needs_layout_passes=False
        ),
        out_type=jax.ShapeDtypeStruct((2, n_tiles, _OUT // 2, _TILE), jnp.float32),
        scratch_types=[
            pltpu.VMEM((epw,), jnp.int32),
            pltpu.VMEM((epw,), jnp.int32),
        ]
        + [pltpu.VMEM((chunk, _OUT), jnp.float32) for _ in range(2)]
        + [pltpu.VMEM((2, _CHUNK_TILES, _OUT // 2, _TILE), jnp.float32) for _ in range(2)]
        + [pltpu.SemaphoreType.DMA for _ in range(7)],
    )
    def _gather(src_hbm, dst_hbm, ei_hbm, out_hbm, u_all, v_all, a0, a1, t0, t1, *sems):
        a_v = [a0, a1]
        t_v = [t0, t1]
        si = sems[0]
        sg1 = list(sems[1:3])
        sg2 = list(sems[3:5])
        sst = list(sems[5:7])

        wid = lax.axis_index("s") * _NUM_CORES + lax.axis_index("c")
        tile_lo = jnp.minimum(wid * n_tiles // _NW, n_tiles - tw)
        base0 = tile_lo * _TILE

        cu = pltpu.async_copy(ei_hbm.at[0, pl.ds(base0, epw)], u_all, si)
        cv = pltpu.async_copy(ei_hbm.at[1, pl.ds(base0, epw)], v_all, si)
        cu.wait()
        cv.wait()

        iota = lax.iota(jnp.int32, _LANE)
        fcols = [jnp.full((_LANE,), f, jnp.int32) for f in range(_OUT)]

        def _transpose(a_ref, t_ref):
            @functools.partial(plsc.parallel_loop, 0, groups, unroll=4)
            def body(g):
                row_idx = g * _LANE + iota
                tile = g // (_TILE // _LANE)
                e_off = (g % (_TILE // _LANE)) * _LANE
                for f in range(_OUT):
                    vec = plsc.load_gather(a_ref, [row_idx, fcols[f]])
                    t_ref[f // 8, tile, f % 8, pl.ds(e_off, _LANE)] = vec

        g1 = [None] * n_chunks
        g2 = [None] * n_chunks
        st = [None] * n_chunks

        g1[0] = pltpu.async_copy(
            src_hbm.at[u_all.at[pl.ds(0, chunk)]], a_v[0], sg1[0]
        )
        for k in range(n_chunks):
            s = k % 2
            g1[k].wait()
            g2[k] = pltpu.async_copy(
                dst_hbm.at[v_all.at[pl.ds(k * chunk, chunk)]],
                a_v[s],
                sg2[s],
                add=True,
            )
            if k + 1 < n_chunks:
                g1[k + 1] = pltpu.async_copy(
                    src_hbm.at[u_all.at[pl.ds((k + 1) * chunk, chunk)]],
                    a_v[(k + 1) % 2],
                    sg1[(k + 1) % 2],
                )
            g2[k].wait()
            if k >= 2:
                st[k - 2].wait()
            _transpose(a_v[s], t_v[s])
            st[k] = pltpu.async_copy(
                t_v[s],
                out_hbm.at[:, pl.ds(tile_lo + k * _CHUNK_TILES, _CHUNK_TILES)],
                sst[s],
            )
        st[n_chunks - 2].wait()
        st[n_chunks - 1].wait()

    return _gather


def kernel(x, edge_index, W_src, b_src, W_dst, b_dst):
    src_tab, dst_tab = _proj(x, W_src, b_src, W_dst, b_dst)
    ei = edge_index.astype(jnp.int32)
    n_edges = ei.shape[1]
    gather = _make_gather(n_edges)
    v = gather(src_tab, dst_tab, ei)
    return v.transpose(1, 3, 0, 2).reshape(n_edges, _OUT)
